# baseline (speedup 1.0000x reference)
import jax
import jax.numpy as jnp
from jax.experimental import pallas as pl
from jax.experimental.pallas import tpu as pltpu

H, W, DH = 32, 32, 384
NCHUNK = 8
CH = H // NCHUNK


def _body(rowv, col_hbm, out_hbm, buf, csems, rsems):
    col_copies = []
    for hh in range(H):
        cp = pltpu.make_async_copy(
            col_hbm.at[pl.ds(0, W)],
            out_hbm.at[pl.ds(W * hh, W), pl.ds(0, DH)],
            csems.at[hh % NCHUNK],
        )
        cp.start()
        col_copies.append(cp)
    row_copies = []
    for k in range(NCHUNK):
        rows = rowv[CH * k:CH * (k + 1), :]
        val = jnp.broadcast_to(rows[:, None, :], (CH, W, DH)).reshape(CH * W, DH)
        buf[CH * W * k:CH * W * (k + 1), :] = val
        cp = pltpu.make_async_copy(
            buf.at[pl.ds(CH * W * k, CH * W)],
            out_hbm.at[pl.ds(CH * W * k, CH * W), pl.ds(DH, DH)],
            rsems.at[k],
        )
        cp.start()
        row_copies.append(cp)
    for cp in col_copies:
        cp.wait()
    for cp in row_copies:
        cp.wait()


def kernel(h, w, row_embed, col_embed):
    out = pl.pallas_call(
        _body,
        grid=(1,),
        in_specs=[
            pl.BlockSpec((H, DH), lambda i: (0, 0)),
            pl.BlockSpec(memory_space=pl.ANY),
        ],
        out_specs=pl.BlockSpec(memory_space=pl.ANY, index_map=lambda i: (0, 0)),
        out_shape=jax.ShapeDtypeStruct((H * W, 2 * DH), jnp.float32),
        scratch_shapes=[
            pltpu.VMEM((H * W, DH), jnp.float32),
            pltpu.SemaphoreType.DMA((NCHUNK,)),
            pltpu.SemaphoreType.DMA((NCHUNK,)),
        ],
    )(row_embed, col_embed)
    return out.reshape(H, W, 2 * DH)


# speedup vs baseline: 18.3553x; 18.3553x over previous
import jax
import jax.numpy as jnp
from jax.experimental import pallas as pl
from jax.experimental.pallas import tpu as pltpu

H, W, DH = 32, 32, 384
NCHUNK = 4
CH = H // NCHUNK


def _body(row_ref, col_ref, out_hbm, buf, sems):
    col = col_ref[0:W, :]
    colb = jnp.broadcast_to(col[None, :, :], (CH, W, DH))
    copies = []
    for k in range(NCHUNK):
        row = row_ref[CH * k:CH * (k + 1), :]
        buf[CH * k:CH * (k + 1), :, 0:DH] = colb
        buf[CH * k:CH * (k + 1), :, DH:2 * DH] = jnp.broadcast_to(
            row[:, None, :], (CH, W, DH))
        cp = pltpu.make_async_copy(
            buf.at[pl.ds(CH * k, CH)],
            out_hbm.at[pl.ds(CH * k, CH)],
            sems.at[k],
        )
        cp.start()
        copies.append(cp)
    for cp in copies:
        cp.wait()


def kernel(h, w, row_embed, col_embed):
    return pl.pallas_call(
        _body,
        in_specs=[
            pl.BlockSpec(memory_space=pltpu.VMEM),
            pl.BlockSpec(memory_space=pltpu.VMEM),
        ],
        out_specs=pl.BlockSpec(memory_space=pl.ANY),
        out_shape=jax.ShapeDtypeStruct((H, W, 2 * DH), jnp.float32),
        scratch_shapes=[
            pltpu.VMEM((H, W, 2 * DH), jnp.float32),
            pltpu.SemaphoreType.DMA((NCHUNK,)),
        ],
    )(row_embed, col_embed)
